# baseline (speedup 1.0000x reference)
import functools

import jax
import jax.numpy as jnp
import numpy as np
from jax import lax
from jax.experimental import pallas as pl
from jax.experimental.pallas import tpu as pltpu
from jax.experimental.pallas import tpu_sc as plsc

B = 16384
C = 2
BC = B // C
EMB = 128
FEAT = 3 * EMB
NC = 2
NS = 16
NW = NC * NS
RPW = BC // NW
SUB = 128
NSUB = RPW // SUB
PWROWS = RPW // 8


def _sc_gather(pk, bs3, user_W, product_W, cluster_Wp, wide_user_W, wide_cat_W):
    mesh = plsc.VectorSubcoreMesh(core_axis_name="c", subcore_axis_name="s")

    @functools.partial(
        pl.kernel,
        mesh=mesh,
        out_type=[
            jax.ShapeDtypeStruct((BC, FEAT), jnp.float32),
            jax.ShapeDtypeStruct((BC // 8, EMB), jnp.float32),
        ],
        scratch_types=[
            pltpu.VMEM((4 * NSUB, SUB), jnp.int32),
            pltpu.VMEM((NSUB, SUB), jnp.float32),
            pltpu.VMEM((RPW, EMB), jnp.float32),
            pltpu.VMEM((SUB, EMB), jnp.float32),
            pltpu.VMEM((SUB, EMB), jnp.float32),
            pltpu.VMEM((SUB, EMB), jnp.float32),
            pltpu.VMEM((SUB, EMB), jnp.float32),
            pltpu.VMEM((PWROWS, EMB), jnp.float32),
            pltpu.SemaphoreType.DMA,
            pltpu.SemaphoreType.DMA,
            pltpu.SemaphoreType.DMA,
            pltpu.SemaphoreType.DMA,
        ],
    )
    def k(pk_hbm, bs_hbm, uW, pW, cW, wuW, wcW,
          feat_o, wide_o,
          pkv, bsv, bu, bp, bc, bwu, bwc, pw, semA, semB, semC, semD):
        wid = lax.axis_index("s") * NC + lax.axis_index("c")
        base = wid * RPW
        s0 = pltpu.async_copy(pk_hbm.at[wid], pkv, semA)
        s1 = pltpu.async_copy(bs_hbm.at[wid], bsv, semA)
        s0.wait()
        s1.wait()

        def idx(plane, j):
            return pkv.at[plane * NSUB + j]

        def wide_half(j):
            def step(q, _):
                for i in range(8):
                    r = q * 8 + i
                    acc = bwu[r, pl.ds(0, 16)] * bwc[r, pl.ds(0, 16)]
                    for g in range(1, EMB // 16):
                        acc = acc + bwu[r, pl.ds(16 * g, 16)] * bwc[r, pl.ds(16 * g, 16)]
                    pw[j * (SUB // 8) + q, pl.ds(i * 16, 16)] = acc
                return _
            lax.fori_loop(0, SUB // 8, step, 0)

        def inject_bs(j):
            for o in range(0, SUB, 16):
                vals = bsv[j, pl.ds(o, 16)]
                for l in range(16):
                    v = jnp.full((16,), vals[l], jnp.float32)
                    bc[o + l, pl.ds(EMB // 2, 16)] = v

        up = [
            pltpu.async_copy(uW.at[idx(0, j)], bu.at[pl.ds(j * SUB, SUB)], semA)
            for j in range(NSUB)
        ]
        pp = pltpu.async_copy(pW.at[idx(1, 0)], bp, semD)
        cc = pltpu.async_copy(cW.at[idx(2, 0)], bc, semB)
        w0 = pltpu.async_copy(wuW.at[idx(0, 0)], bwu, semC)
        w1 = pltpu.async_copy(wcW.at[idx(3, 0)], bwc, semC)
        w0.wait()
        w1.wait()
        wide_half(0)
        w0 = pltpu.async_copy(wuW.at[idx(0, 1)], bwu, semC)
        w1 = pltpu.async_copy(wcW.at[idx(3, 1)], bwc, semC)
        cc.wait()
        inject_bs(0)
        pltpu.sync_copy(bc, feat_o.at[pl.ds(base, SUB), pl.ds(2 * EMB, EMB)])
        cc = pltpu.async_copy(cW.at[idx(2, 1)], bc, semB)
        pp.wait()
        pltpu.sync_copy(bp, feat_o.at[pl.ds(base, SUB), pl.ds(EMB, EMB)])
        pp = pltpu.async_copy(pW.at[idx(1, 1)], bp, semD)
        for cp in up:
            cp.wait()
        ou = pltpu.async_copy(bu, feat_o.at[pl.ds(base, RPW), pl.ds(0, EMB)], semA)
        w0.wait()
        w1.wait()
        wide_half(1)
        cc.wait()
        inject_bs(1)
        pltpu.sync_copy(bc, feat_o.at[pl.ds(base + SUB, SUB), pl.ds(2 * EMB, EMB)])
        pp.wait()
        pltpu.sync_copy(bp, feat_o.at[pl.ds(base + SUB, SUB), pl.ds(EMB, EMB)])
        pltpu.sync_copy(pw, wide_o.at[pl.ds(wid * PWROWS, PWROWS)])
        ou.wait()

    return k(pk, bs3, user_W, product_W, cluster_Wp, wide_user_W, wide_cat_W)


def _xwt(x, w):
    return lax.dot_general(x, w, (((1,), (1,)), ((), ())),
                           preferred_element_type=jnp.float32)


def _tc_body(feat_r, w16_r,
             B1_r, b1_r, B2_r, b2_r, B3_r, b3_r, a6_r, b4_r,
             o_r):
    bf16 = jnp.bfloat16
    acc = _xwt(feat_r[...].astype(bf16), B1_r[...]) + b1_r[...]
    h = jnp.maximum(acc, 0.0).astype(bf16)
    h = jnp.maximum(_xwt(h, B2_r[...]) + b2_r[...], 0.0).astype(bf16)
    h = jnp.maximum(_xwt(h, B3_r[...]) + b3_r[...], 0.0)
    logit = jnp.sum(h * a6_r[...], axis=1) + b4_r[0, 0]
    bm8 = w16_r.shape[0]
    wide = jnp.sum(w16_r[...].reshape(bm8, 8, 16), axis=2).reshape(bm8 * 8)
    o_r[...] = 1.0 / (1.0 + jnp.exp(-(logit + wide)))


def _tc_mlp(feat, w16, B1, b1s, B2, b2s, B3, b3s, a6, b4r):
    bm = 2048
    grid = (BC // bm,)

    def blk(shape):
        return pl.BlockSpec(shape, lambda i: (i, 0))

    def full(a):
        return pl.BlockSpec(a.shape, lambda i: (0,) * a.ndim)

    return pl.pallas_call(
        _tc_body,
        grid=grid,
        in_specs=[
            blk((bm, FEAT)), blk((bm // 8, EMB)),
            full(B1), full(b1s), full(B2), full(b2s), full(B3), full(b3s),
            full(a6), full(b4r),
        ],
        out_specs=pl.BlockSpec((bm,), lambda i: (i,)),
        out_shape=jax.ShapeDtypeStruct((BC,), jnp.float32),
    )(feat, w16, B1, b1s, B2, b2s, B3, b3s, a6, b4r)


def kernel(user_ids, product_ids, category_ids, cluster_ids, behavior_scores,
           wide_user_W, wide_cat_W, user_W, product_W, cluster_W,
           W1, b1, W2, b2, W3, b3, W4, b4):
    pk_all = jnp.stack([
        user_ids.astype(jnp.int32),
        product_ids.astype(jnp.int32),
        cluster_ids.astype(jnp.int32),
        category_ids.astype(jnp.int32),
    ]).reshape(4, C, NW, NSUB, SUB).transpose(1, 2, 0, 3, 4).reshape(
        C, NW, 4 * NSUB, SUB)
    bs4 = behavior_scores.reshape(C, NW, NSUB, SUB)

    cluster_Wp = jnp.pad(cluster_W, ((0, 0), (0, EMB // 2)))

    s = float(1.0 / np.sqrt(1.0 + 1e-5))
    bf16 = jnp.bfloat16
    B1 = jnp.pad(W1 * s, ((0, 0), (0, FEAT - W1.shape[1]))).astype(bf16)
    b1s = (b1 * s).reshape(1, -1)
    B2 = (W2 * s).astype(bf16)
    b2s = (b2 * s).reshape(1, -1)
    B3 = (W3 * s).astype(bf16)
    b3s = (b3 * s).reshape(1, -1)
    a6 = W4.reshape(1, -1)
    b4r = b4.reshape(1, 1)

    outs = []
    for c in range(C):
        feat, w16 = _sc_gather(
            pk_all[c], bs4[c], user_W, product_W, cluster_Wp,
            wide_user_W, wide_cat_W)
        outs.append(_tc_mlp(feat, w16, B1, b1s, B2, b2s, B3, b3s, a6, b4r))
    return jnp.concatenate(outs)


# speedup vs baseline: 1.0025x; 1.0025x over previous
import functools

import jax
import jax.numpy as jnp
import numpy as np
from jax import lax
from jax.experimental import pallas as pl
from jax.experimental.pallas import tpu as pltpu
from jax.experimental.pallas import tpu_sc as plsc

B = 16384
C = 2
BC = B // C
EMB = 128
FEAT = 3 * EMB
NC = 2
NS = 16
NW = NC * NS
RPW = BC // NW
SUB = 128
NSUB = RPW // SUB
PWROWS = RPW // 8


def _sc_gather(pk, bs3, user_W, product_W, cluster_Wp, wide_user_W, wide_cat_W):
    mesh = plsc.VectorSubcoreMesh(core_axis_name="c", subcore_axis_name="s")

    @functools.partial(
        pl.kernel,
        mesh=mesh,
        out_type=[
            jax.ShapeDtypeStruct((BC, FEAT), jnp.float32),
            jax.ShapeDtypeStruct((BC // 8, EMB), jnp.float32),
        ],
        scratch_types=[
            pltpu.VMEM((4 * NSUB, SUB), jnp.int32),
            pltpu.VMEM((NSUB, SUB), jnp.float32),
            pltpu.VMEM((RPW, EMB), jnp.float32),
            pltpu.VMEM((SUB, EMB), jnp.float32),
            pltpu.VMEM((SUB, EMB), jnp.float32),
            pltpu.VMEM((SUB, EMB), jnp.float32),
            pltpu.VMEM((SUB, EMB), jnp.float32),
            pltpu.VMEM((PWROWS, EMB), jnp.float32),
            pltpu.SemaphoreType.DMA,
            pltpu.SemaphoreType.DMA,
            pltpu.SemaphoreType.DMA,
            pltpu.SemaphoreType.DMA,
        ],
    )
    def k(pk_hbm, bs_hbm, uW, pW, cW, wuW, wcW,
          feat_o, wide_o,
          pkv, bsv, bu, bp, bc, bwu, bwc, pw, semA, semB, semC, semD):
        wid = lax.axis_index("s") * NC + lax.axis_index("c")
        base = wid * RPW
        s0 = pltpu.async_copy(pk_hbm.at[wid], pkv, semA)
        s1 = pltpu.async_copy(bs_hbm.at[wid], bsv, semA)
        s0.wait()
        s1.wait()

        def idx(plane, j):
            return pkv.at[plane * NSUB + j]

        def wide_half(j):
            def step(q, _):
                for i in range(8):
                    r = q * 8 + i
                    acc = bwu[r, pl.ds(0, 16)] * bwc[r, pl.ds(0, 16)]
                    for g in range(1, EMB // 16):
                        acc = acc + bwu[r, pl.ds(16 * g, 16)] * bwc[r, pl.ds(16 * g, 16)]
                    pw[j * (SUB // 8) + q, pl.ds(i * 16, 16)] = acc
                return _
            lax.fori_loop(0, SUB // 8, step, 0)

        def inject_bs(j):
            for o in range(0, SUB, 16):
                vals = bsv[j, pl.ds(o, 16)]
                for l in range(16):
                    v = jnp.full((16,), vals[l], jnp.float32)
                    bc[o + l, pl.ds(EMB // 2, 16)] = v

        up = [
            pltpu.async_copy(uW.at[idx(0, j)], bu.at[pl.ds(j * SUB, SUB)], semA)
            for j in range(NSUB)
        ]
        pp = pltpu.async_copy(pW.at[idx(1, 0)], bp, semD)
        cc = pltpu.async_copy(cW.at[idx(2, 0)], bc, semB)
        w0 = pltpu.async_copy(wuW.at[idx(0, 0)], bwu, semC)
        w1 = pltpu.async_copy(wcW.at[idx(3, 0)], bwc, semC)
        w0.wait()
        w1.wait()
        wide_half(0)
        w0 = pltpu.async_copy(wuW.at[idx(0, 1)], bwu, semC)
        w1 = pltpu.async_copy(wcW.at[idx(3, 1)], bwc, semC)
        cc.wait()
        inject_bs(0)
        pltpu.sync_copy(bc, feat_o.at[pl.ds(base, SUB), pl.ds(2 * EMB, EMB)])
        cc = pltpu.async_copy(cW.at[idx(2, 1)], bc, semB)
        pp.wait()
        pltpu.sync_copy(bp, feat_o.at[pl.ds(base, SUB), pl.ds(EMB, EMB)])
        pp = pltpu.async_copy(pW.at[idx(1, 1)], bp, semD)
        for cp in up:
            cp.wait()
        ou = pltpu.async_copy(bu, feat_o.at[pl.ds(base, RPW), pl.ds(0, EMB)], semA)
        w0.wait()
        w1.wait()
        wide_half(1)
        cc.wait()
        inject_bs(1)
        pltpu.sync_copy(bc, feat_o.at[pl.ds(base + SUB, SUB), pl.ds(2 * EMB, EMB)])
        pp.wait()
        pltpu.sync_copy(bp, feat_o.at[pl.ds(base + SUB, SUB), pl.ds(EMB, EMB)])
        pltpu.sync_copy(pw, wide_o.at[pl.ds(wid * PWROWS, PWROWS)])
        ou.wait()

    return k(pk, bs3, user_W, product_W, cluster_Wp, wide_user_W, wide_cat_W)


def _xwt(x, w):
    return lax.dot_general(x, w, (((1,), (1,)), ((), ())),
                           preferred_element_type=jnp.float32)


def _tc_body(feat_r, w16_r,
             B1_r, b1_r, B2_r, b2_r, B3_r, b3_r, a6_r, b4_r,
             o_r):
    bf16 = jnp.bfloat16
    acc = _xwt(feat_r[...].astype(bf16), B1_r[...]) + b1_r[...]
    h = jnp.maximum(acc, 0.0).astype(bf16)
    h = jnp.maximum(_xwt(h, B2_r[...]) + b2_r[...], 0.0).astype(bf16)
    h = jnp.maximum(_xwt(h, B3_r[...]) + b3_r[...], 0.0)
    logit = jnp.sum(h * a6_r[...], axis=1) + b4_r[0, 0]
    bm8 = w16_r.shape[0]
    wide = jnp.sum(w16_r[...].reshape(bm8, 8, 16), axis=2).reshape(bm8 * 8)
    o_r[...] = 1.0 / (1.0 + jnp.exp(-(logit + wide)))


def _tc_mlp(feat, w16, B1, b1s, B2, b2s, B3, b3s, a6, b4r):
    bm = 1024
    grid = (BC // bm,)

    def blk(shape):
        return pl.BlockSpec(shape, lambda i: (i, 0))

    def full(a):
        return pl.BlockSpec(a.shape, lambda i: (0,) * a.ndim)

    return pl.pallas_call(
        _tc_body,
        grid=grid,
        in_specs=[
            blk((bm, FEAT)), blk((bm // 8, EMB)),
            full(B1), full(b1s), full(B2), full(b2s), full(B3), full(b3s),
            full(a6), full(b4r),
        ],
        out_specs=pl.BlockSpec((bm,), lambda i: (i,)),
        out_shape=jax.ShapeDtypeStruct((BC,), jnp.float32),
    )(feat, w16, B1, b1s, B2, b2s, B3, b3s, a6, b4r)


def kernel(user_ids, product_ids, category_ids, cluster_ids, behavior_scores,
           wide_user_W, wide_cat_W, user_W, product_W, cluster_W,
           W1, b1, W2, b2, W3, b3, W4, b4):
    pk_all = jnp.stack([
        user_ids.astype(jnp.int32),
        product_ids.astype(jnp.int32),
        cluster_ids.astype(jnp.int32),
        category_ids.astype(jnp.int32),
    ]).reshape(4, C, NW, NSUB, SUB).transpose(1, 2, 0, 3, 4).reshape(
        C, NW, 4 * NSUB, SUB)
    bs4 = behavior_scores.reshape(C, NW, NSUB, SUB)

    cluster_Wp = jnp.pad(cluster_W, ((0, 0), (0, EMB // 2)))

    s = float(1.0 / np.sqrt(1.0 + 1e-5))
    bf16 = jnp.bfloat16
    B1 = jnp.pad(W1 * s, ((0, 0), (0, FEAT - W1.shape[1]))).astype(bf16)
    b1s = (b1 * s).reshape(1, -1)
    B2 = (W2 * s).astype(bf16)
    b2s = (b2 * s).reshape(1, -1)
    B3 = (W3 * s).astype(bf16)
    b3s = (b3 * s).reshape(1, -1)
    a6 = W4.reshape(1, -1)
    b4r = b4.reshape(1, 1)

    outs = []
    for c in range(C):
        feat, w16 = _sc_gather(
            pk_all[c], bs4[c], user_W, product_W, cluster_Wp,
            wide_user_W, wide_cat_W)
        outs.append(_tc_mlp(feat, w16, B1, b1s, B2, b2s, B3, b3s, a6, b4r))
    return jnp.concatenate(outs)
